# Initial kernel scaffold; baseline (speedup 1.0000x reference)
import jax
import jax.numpy as jnp
from jax.experimental import pallas as pl


def kernel(x, edge_index, edge_attr, hidden_state, cell_state, time_series, params):
    raise NotImplementedError("write your pallas kernel here")


# speedup vs baseline: 11.9875x; 11.9875x over previous
import functools

import jax
import jax.numpy as jnp
from jax import lax
from jax.experimental import pallas as pl
from jax.experimental.pallas import tpu as pltpu
from jax.experimental.pallas import tpu_sc as plsc

N = 10000
E = 320000
D = 128
H = 64
K = 100
T = 512
ROI = 200

NC = 2
NS = 16
NW = NC * NS
CH = 80
RPT = 624
RTAIL = N - NS * RPT


def _sliced_copy(copy_one, sid):
    copy_one(sid * RPT, RPT)

    @pl.when(sid == NS - 1)
    def _():
        copy_one(NS * RPT, RTAIL)

_mesh = plsc.VectorSubcoreMesh(core_axis_name="c", subcore_axis_name="s")
_sc_params = pltpu.CompilerParams(needs_layout_passes=False)


@functools.partial(
    pl.kernel,
    mesh=_mesh,
    compiler_params=_sc_params,
    out_type=jax.ShapeDtypeStruct((NC, N, D), jnp.float32),
    scratch_types=[
        pltpu.VMEM_SHARED((N, D), jnp.float32),
        pltpu.VMEM((CH,), jnp.int32),
        pltpu.VMEM((CH, D), jnp.float32),
    ],
)
def _sc_deg(dst_hbm, zeros_hbm, ones_hbm, out_hbm, acc, dst_v, ones_v):
    cid = lax.axis_index("c")
    sid = lax.axis_index("s")

    def zero_one(r0, nr):
        r0 = pl.multiple_of(r0, 8)
        pltpu.sync_copy(zeros_hbm.at[pl.ds(r0, nr)], acc.at[pl.ds(r0, nr)])

    _sliced_copy(zero_one, sid)
    pltpu.sync_copy(ones_hbm, ones_v)
    plsc.subcore_barrier()

    per_worker = E // NW
    base = (cid * NS + sid) * per_worker

    def body(k, carry):
        off = pl.multiple_of(base + k * CH, 8)
        pltpu.sync_copy(dst_hbm.at[pl.ds(off, CH)], dst_v)
        pltpu.sync_copy(ones_v, acc.at[dst_v], add=True)
        return carry

    lax.fori_loop(0, per_worker // CH, body, 0)
    plsc.subcore_barrier()

    def wb_one(r0, nr):
        r0 = pl.multiple_of(r0, 8)
        pltpu.sync_copy(acc.at[pl.ds(r0, nr)], out_hbm.at[cid, pl.ds(r0, nr)])

    _sliced_copy(wb_one, sid)


@functools.partial(
    pl.kernel,
    mesh=_mesh,
    compiler_params=_sc_params,
    out_type=(jax.ShapeDtypeStruct((N, D), jnp.float32),
              jax.ShapeDtypeStruct((N, D), jnp.float32)),
    scratch_types=[
        pltpu.VMEM_SHARED((N, D), jnp.float32),
        pltpu.VMEM((CH,), jnp.int32),
        pltpu.VMEM((CH,), jnp.int32),
        pltpu.VMEM((CH,), jnp.float32),
        pltpu.VMEM((CH, D), jnp.float32),
        pltpu.SemaphoreType.DMA,
    ],
)
def _sc_agg(xn_hbm, hd_hbm, src_hbm, dst_hbm, ew_hbm, zd_hbm,
            outA_hbm, outS_hbm,
            acc, src_v, dst_v, ew_v, rows, sem):
    cid = lax.axis_index("c")
    sid = lax.axis_index("s")

    def zero_one(r0, nr):
        r0 = pl.multiple_of(r0, 8)
        pltpu.sync_copy(zd_hbm.at[pl.ds(r0, nr)], acc.at[pl.ds(r0, nr)])

    _sliced_copy(zero_one, sid)
    plsc.subcore_barrier()

    per_tile = E // NS
    base = sid * per_tile

    def chunk(k, carry):
        off = pl.multiple_of(base + k * CH, 8)
        pltpu.sync_copy(src_hbm.at[pl.ds(off, CH)], src_v)
        pltpu.sync_copy(dst_hbm.at[pl.ds(off, CH)], dst_v)

        @pl.when(cid == 0)
        def _():
            pltpu.sync_copy(ew_hbm.at[pl.ds(off, CH)], ew_v)
            pltpu.async_copy(xn_hbm.at[src_v], rows, sem).wait()

            def scale(e, c):
                w = plsc.load_gather(ew_v, [jnp.full((16,), e, jnp.int32)])
                for j in range(D // 16):
                    rows[e, pl.ds(j * 16, 16)] = rows[e, pl.ds(j * 16, 16)] * w
                return c

            lax.fori_loop(0, CH, scale, 0)

        @pl.when(cid == 1)
        def _():
            pltpu.async_copy(hd_hbm.at[src_v], rows, sem).wait()

        pltpu.sync_copy(rows, acc.at[dst_v], add=True)
        return carry

    lax.fori_loop(0, per_tile // CH, chunk, 0)
    plsc.subcore_barrier()

    def wb_one(r0, nr):
        r0 = pl.multiple_of(r0, 8)

        @pl.when(cid == 0)
        def _():
            pltpu.sync_copy(acc.at[pl.ds(r0, nr)], outA_hbm.at[pl.ds(r0, nr)])

        @pl.when(cid == 1)
        def _():
            pltpu.sync_copy(acc.at[pl.ds(r0, nr)], outS_hbm.at[pl.ds(r0, nr)])

    _sliced_copy(wb_one, sid)


def _tc_prep_body(x_ref, h_ref, degs_ref, xn_ref, hd_ref, dis_ref):
    x = x_ref[...]
    mean = jnp.mean(x, axis=0, keepdims=True)
    var1 = jnp.sum((x - mean) ** 2, axis=0, keepdims=True) / (N - 1)
    xn = (x - mean) / (jnp.sqrt(var1) + 1e-6)
    xn_ref[...] = xn
    deg = degs_ref[0] + degs_ref[1] + 1.0
    dis = lax.rsqrt(deg)
    dis_ref[...] = dis
    hd_ref[:, 0:H] = dis * h_ref[...]
    hd_ref[:, H:D] = jnp.zeros((N, H), jnp.float32)


def _tc_prep(x, hidden, degs):
    return pl.pallas_call(
        _tc_prep_body,
        out_shape=(jax.ShapeDtypeStruct((N, D), jnp.float32),
                   jax.ShapeDtypeStruct((N, D), jnp.float32),
                   jax.ShapeDtypeStruct((N, 1), jnp.float32)),
    )(x, hidden, degs)


def _tc_b1_body(A_ref, xn_ref, S_ref, h_ref, c_ref, dis_ref,
                wrel_ref, wroot_ref, wh_ref, b_ref, v_ref,
                hn_ref, sraw_ref):
    dis = dis_ref[...]
    C = dis * S_ref[:, 0:H] + (dis * dis) * h_ref[...]
    Z = (jnp.dot(A_ref[...], wrel_ref[...], preferred_element_type=jnp.float32)
         + jnp.dot(xn_ref[...], wroot_ref[...], preferred_element_type=jnp.float32)
         + jnp.dot(C, wh_ref[...], preferred_element_type=jnp.float32)
         + b_ref[...])
    ig = jax.nn.sigmoid(Z[:, 0:H])
    fg = jax.nn.sigmoid(Z[:, H:2 * H])
    og = jax.nn.sigmoid(Z[:, 2 * H:3 * H])
    mod = jax.nn.relu(Z[:, 3 * H:4 * H])
    cell = jnp.tanh(ig * mod + fg * c_ref[...])
    hn = og * jnp.tanh(cell)
    hn_ref[...] = hn
    v = v_ref[...]
    vhat = v / (jnp.sqrt(jnp.sum(v * v)) + 1e-8)
    sraw_ref[...] = jnp.dot(hn, vhat, preferred_element_type=jnp.float32)


def _tc_b1(A, xn, S, hidden, cell, dis, Wrel, Wroot, Wh, b, v):
    BR = 2000
    grid = (N // BR,)
    row = lambda i: (i, 0)
    rep = lambda i: (0, 0)
    return pl.pallas_call(
        _tc_b1_body,
        grid=grid,
        in_specs=[
            pl.BlockSpec((BR, D), row), pl.BlockSpec((BR, D), row),
            pl.BlockSpec((BR, D), row), pl.BlockSpec((BR, H), row),
            pl.BlockSpec((BR, H), row), pl.BlockSpec((BR, 1), row),
            pl.BlockSpec((D, 4 * H), rep), pl.BlockSpec((D, 4 * H), rep),
            pl.BlockSpec((H, 4 * H), rep), pl.BlockSpec((1, 4 * H), rep),
            pl.BlockSpec((H, 1), rep),
        ],
        out_specs=[pl.BlockSpec((BR, H), row), pl.BlockSpec((BR, 1), row)],
        out_shape=(jax.ShapeDtypeStruct((N, H), jnp.float32),
                   jax.ShapeDtypeStruct((N, 1), jnp.float32)),
    )(A, xn, S, hidden, cell, dis, Wrel, Wroot, Wh, b, v)


def _tc_lstm_body(ts_ref, wih_t_ref, whh_t_ref, bih_ref, bhh_ref, out_ref,
                  g_ref):
    g_ref[...] = jnp.dot(ts_ref[...], wih_t_ref[...],
                         preferred_element_type=jnp.float32) + bih_ref[...]
    whh_t = whh_t_ref[...]
    bhh = bhh_ref[...]

    def step(t, hc):
        h, c = hc
        g = g_ref[pl.ds(t, 1), :] + jnp.dot(h, whh_t,
                                            preferred_element_type=jnp.float32) + bhh
        i = jax.nn.sigmoid(g[:, 0:H])
        f = jax.nn.sigmoid(g[:, H:2 * H])
        gg = jnp.tanh(g[:, 2 * H:3 * H])
        o = jax.nn.sigmoid(g[:, 3 * H:4 * H])
        c = f * c + i * gg
        h = o * jnp.tanh(c)
        return (h, c)

    h0 = jnp.zeros((1, H), jnp.float32)
    h, _ = lax.fori_loop(0, T, step, (h0, h0))
    out_ref[...] = h


def _tc_lstm(ts, wih_t, whh_t, bih, bhh):
    return pl.pallas_call(
        _tc_lstm_body,
        out_shape=jax.ShapeDtypeStruct((1, H), jnp.float32),
        scratch_shapes=[pltpu.VMEM((T, 4 * H), jnp.float32)],
    )(ts, wih_t, whh_t, bih, bhh)


def _tc_b2_body(hn_ref, sraw_ref, low_ref, lng_ref, lnb_ref,
                w1_ref, b1_ref, w2_ref, b2_ref,
                scores_ref, pool_ref, pred_ref):
    s = sraw_ref[...]
    mu = jnp.mean(s)
    sd = jnp.sqrt(jnp.mean((s - mu) ** 2))
    scores = (s - mu) / (sd + 1e-8)
    scores_ref[...] = scores
    sig = jax.nn.sigmoid(scores)
    pool_ref[...] = jnp.mean(sig * (1.0 - sig)).reshape(1, 1)

    bits = lax.bitcast_convert_type(scores, jnp.uint32)
    neg = (bits & jnp.uint32(0x80000000)) != 0
    y = jnp.where(neg, ~bits, bits | jnp.uint32(0x80000000))

    def tbody(i, t):
        cand = t | (jnp.uint32(1) << jnp.uint32(31 - i))
        cnt = jnp.sum((y >= cand).astype(jnp.float32))
        return jnp.where(cnt >= K, cand, t)

    t = lax.fori_loop(0, 32, tbody, jnp.uint32(0))
    n_gt = jnp.sum((y > t).astype(jnp.float32))
    r = jnp.float32(K) - n_gt
    idx = lax.broadcasted_iota(jnp.int32, (N, 1), 0)
    ties = y == t

    def jbody(i, jj):
        cand = jj | (1 << (13 - i))
        c2 = jnp.sum((ties & (idx < cand)).astype(jnp.float32))
        return jnp.where(c2 < r, cand, jj)

    jj = lax.fori_loop(0, 14, jbody, jnp.int32(0))
    sel = (y > t) | (ties & (idx <= jj) & (r > 0))
    m = sel.astype(jnp.float32)
    high = jnp.sum(m * (hn_ref[...] * sig), axis=0, keepdims=True) / K

    fusion = jnp.concatenate([high, low_ref[...]], axis=1)
    fmu = jnp.mean(fusion)
    fvar = jnp.mean((fusion - fmu) ** 2)
    fusion = (fusion - fmu) / jnp.sqrt(fvar + 1e-5) * lng_ref[...] + lnb_ref[...]
    h1 = jax.nn.relu(jnp.dot(fusion, w1_ref[...],
                             preferred_element_type=jnp.float32) + b1_ref[...])
    pred_ref[...] = jnp.dot(h1, w2_ref[...],
                            preferred_element_type=jnp.float32) + b2_ref[...]


def _tc_b2(hn, sraw, low, lng, lnb, w1, b1, w2, b2):
    return pl.pallas_call(
        _tc_b2_body,
        out_shape=(jax.ShapeDtypeStruct((N, 1), jnp.float32),
                   jax.ShapeDtypeStruct((1, 1), jnp.float32),
                   jax.ShapeDtypeStruct((1, 1), jnp.float32)),
    )(hn, sraw, low, lng, lnb, w1, b1, w2, b2)


def kernel(x, edge_index, edge_attr, hidden_state, cell_state, time_series,
           params):
    p = params
    src = edge_index[0]
    dst = edge_index[1]

    zeros_nd = jnp.zeros((N, D), jnp.float32)
    degs = _sc_deg(dst, zeros_nd, jnp.ones((CH, D), jnp.float32))
    xn, hd, dis = _tc_prep(x, hidden_state, degs[:, :, 0:1])
    A, S = _sc_agg(xn, hd, src, dst, edge_attr, zeros_nd)

    low = _tc_lstm(time_series, p['Wih'].T, p['Whh'].T,
                   p['bih'].reshape(1, 4 * H), p['bhh'].reshape(1, 4 * H))

    names = ['input', 'forget', 'output', 'modulation']
    Wrel = jnp.concatenate([p[n + '_Wrel'] for n in names], axis=1)
    Wroot = jnp.concatenate([p[n + '_Wroot'] for n in names], axis=1)
    Wh = jnp.concatenate([p[n + '_Wh'] for n in names], axis=1)
    b = jnp.concatenate([p[n + '_brel'] + p[n + '_bh'] for n in names]
                        ).reshape(1, 4 * H)

    hn, sraw = _tc_b1(A, xn, S, hidden_state, cell_state, dis,
                      Wrel, Wroot, Wh, b, p['pool_v'])
    scores, pool, pred = _tc_b2(hn, sraw, low,
                                p['ln_g'].reshape(1, 2 * H),
                                p['ln_b'].reshape(1, 2 * H),
                                p['W1'], p['b1'].reshape(1, H),
                                p['W2'], p['b2'].reshape(1, 1))
    return pred.reshape(1), scores, pool.reshape(())
